# Initial kernel scaffold; baseline (speedup 1.0000x reference)
import jax
import jax.numpy as jnp
from jax.experimental import pallas as pl


def kernel(test, question, tag, correct, mask, interaction, elapsed, emb_interaction, emb_test, emb_question, emb_tag, W_comb, b_comb, g_comb, be_comb, W_cont, b_cont, g_cont, be_cont):
    raise NotImplementedError("write your pallas kernel here")


# speedup vs baseline: 1.0343x; 1.0343x over previous
import functools

import jax
import jax.numpy as jnp
from jax import lax
from jax.experimental import pallas as pl
from jax.experimental.pallas import tpu as pltpu
from jax.experimental.pallas import tpu_sc as plsc

B, L = 4096, 200
BL = B * L
HD2 = 48
EPS = 1e-5

_NC, _NS = 2, 16
NW = _NC * _NS
PER_W = BL // NW
CH = 128
NCHUNK = PER_W // CH

QBLK = 2048


def _proj_q_body(eq_ref, w_ref, out_ref):
    out_ref[...] = jnp.dot(eq_ref[...], w_ref[64:96, :],
                           preferred_element_type=jnp.float32)


def _proj_small_body(et_ref, eg_ref, ei_ref, w_ref, bcomb_ref, gcomb_ref,
                     becomb_ref, wcont_ref, bcont_ref, gcont_ref, becont_ref,
                     pt_ref, pg_ref, pi_ref, cst_ref):
    w = w_ref[...]
    pt_ref[...] = jnp.dot(et_ref[...], w[32:64, :],
                          preferred_element_type=jnp.float32)
    pg_ref[...] = jnp.dot(eg_ref[...], w[96:128, :],
                          preferred_element_type=jnp.float32)
    pi_ref[...] = jnp.dot(ei_ref[...], w[0:32, :],
                          preferred_element_type=jnp.float32) + bcomb_ref[...]
    wc = wcont_ref[...]
    bc = bcont_ref[...]
    gc = gcont_ref[...]
    bec = becont_ref[...]
    A = wc - jnp.mean(wc)
    Bv = bc - jnp.mean(bc)
    Ag = A * gc
    Bg = Bv * gc
    s0 = jnp.full((1, 16), jnp.mean(A * A), jnp.float32)
    s1 = jnp.full((1, 16), 2.0 * jnp.mean(A * Bv), jnp.float32)
    s2 = jnp.full((1, 16), jnp.mean(Bv * Bv) + EPS, jnp.float32)
    cst_ref[...] = jnp.concatenate(
        [gcomb_ref[...], becomb_ref[...], Ag, Bg, bec, s0, s1, s2], axis=1)


def _rsqrt16(x):
    i = lax.bitcast_convert_type(x, jnp.int32)
    y = lax.bitcast_convert_type(
        jnp.int32(0x5F3759DF) - lax.shift_right_logical(i, 1), jnp.float32)
    for _ in range(3):
        y = y * (1.5 - 0.5 * x * y * y)
    return y


def _sc_body(qidx, tidx, gidx, iidx, el, pq, pt, pg, pi, cst, out,
             qi_v, ti_v, gi_v, ii_v, el_v, qr_v, tr_v, gr_v, ir_v, ob_v,
             cst_v, sem):
    c = lax.axis_index("c")
    s = lax.axis_index("s")
    wid = s * _NC + c
    base0 = wid * PER_W

    pltpu.sync_copy(cst, cst_v)
    g0 = cst_v[pl.ds(0, 16)]
    g1 = cst_v[pl.ds(16, 16)]
    g2 = cst_v[pl.ds(32, 16)]
    be0 = cst_v[pl.ds(48, 16)]
    be1 = cst_v[pl.ds(64, 16)]
    be2 = cst_v[pl.ds(80, 16)]
    ag0 = cst_v[pl.ds(96, 16)]
    ag1 = cst_v[pl.ds(112, 16)]
    ag2 = cst_v[pl.ds(128, 16)]
    bg0 = cst_v[pl.ds(144, 16)]
    bg1 = cst_v[pl.ds(160, 16)]
    bg2 = cst_v[pl.ds(176, 16)]
    bec0 = cst_v[pl.ds(192, 16)]
    bec1 = cst_v[pl.ds(208, 16)]
    bec2 = cst_v[pl.ds(224, 16)]
    s0c = cst_v[pl.ds(240, 16)]
    s1c = cst_v[pl.ds(256, 16)]
    s2c = cst_v[pl.ds(272, 16)]

    def chunk(k, carry):
        base = base0 + k * CH
        pltpu.sync_copy(qidx.at[pl.ds(base, CH)], qi_v)
        pltpu.sync_copy(tidx.at[pl.ds(base, CH)], ti_v)
        pltpu.sync_copy(gidx.at[pl.ds(base, CH)], gi_v)
        pltpu.sync_copy(iidx.at[pl.ds(base, CH)], ii_v)
        pltpu.sync_copy(el.at[pl.ds(base, CH)], el_v)
        cp1 = pltpu.async_copy(pq.at[qi_v], qr_v, sem)
        cp2 = pltpu.async_copy(pt.at[ti_v], tr_v, sem)
        cp3 = pltpu.async_copy(pg.at[gi_v], gr_v, sem)
        cp4 = pltpu.async_copy(pi.at[ii_v], ir_v, sem)
        cp1.wait()
        cp2.wait()
        cp3.wait()
        cp4.wait()

        def pos(p, carry2):
            q0 = qr_v[p, pl.ds(0, 16)]
            q1 = qr_v[p, pl.ds(16, 16)]
            q2 = qr_v[p, pl.ds(32, 16)]
            t0 = tr_v[p, pl.ds(0, 16)]
            t1 = tr_v[p, pl.ds(16, 16)]
            t2 = tr_v[p, pl.ds(32, 16)]
            a0 = gr_v[p, pl.ds(0, 16)]
            a1 = gr_v[p, pl.ds(16, 16)]
            a2 = gr_v[p, pl.ds(32, 16)]
            i0 = ir_v[p, pl.ds(0, 16)]
            i1 = ir_v[p, pl.ds(16, 16)]
            i2 = ir_v[p, pl.ds(32, 16)]
            x0 = q0 + t0 + a0 + i0
            x1 = q1 + t1 + a1 + i1
            x2 = q2 + t2 + a2 + i2
            tot = x0 + x1 + x2
            ssum = jnp.sum(tot)
            sq = x0 * x0 + x1 * x1 + x2 * x2
            sqsum = jnp.sum(sq)
            s_v = jnp.full((16,), ssum, jnp.float32)
            sq_v = jnp.full((16,), sqsum, jnp.float32)
            mu = s_v * (1.0 / HD2)
            var = sq_v * (1.0 / HD2) - mu * mu + EPS
            r = _rsqrt16(var)
            ob_v[p, pl.ds(0, 16)] = (x0 - mu) * r * g0 + be0
            ob_v[p, pl.ds(16, 16)] = (x1 - mu) * r * g1 + be1
            ob_v[p, pl.ds(32, 16)] = (x2 - mu) * r * g2 + be2
            a_v = plsc.load_gather(el_v, [jnp.full((16,), p, jnp.int32)])
            varc = a_v * a_v * s0c + a_v * s1c + s2c
            rc = _rsqrt16(varc)
            u = a_v * rc
            ob_v[p, pl.ds(48, 16)] = u * ag0 + rc * bg0 + bec0
            ob_v[p, pl.ds(64, 16)] = u * ag1 + rc * bg1 + bec1
            ob_v[p, pl.ds(80, 16)] = u * ag2 + rc * bg2 + bec2
            return carry2

        lax.fori_loop(0, CH, pos, 0)
        pltpu.sync_copy(ob_v, out.at[pl.ds(base, CH), :])
        return carry

    lax.fori_loop(0, NCHUNK, chunk, 0)


def kernel(test, question, tag, correct, mask, interaction, elapsed,
           emb_interaction, emb_test, emb_question, emb_tag,
           W_comb, b_comb, g_comb, be_comb,
           W_cont, b_cont, g_cont, be_cont):
    nq = emb_question.shape[0]
    nq_blocks = (nq + QBLK - 1) // QBLK
    pq = pl.pallas_call(
        _proj_q_body,
        grid=(nq_blocks,),
        in_specs=[pl.BlockSpec((QBLK, 32), lambda i: (i, 0)),
                  pl.BlockSpec((128, HD2), lambda i: (0, 0))],
        out_specs=pl.BlockSpec((QBLK, HD2), lambda i: (i, 0)),
        out_shape=jax.ShapeDtypeStruct((nq, HD2), jnp.float32),
    )(emb_question, W_comb)

    ei_pad = jnp.zeros((8, 32), jnp.float32).at[0:3, :].set(emb_interaction)
    row = lambda v: v.reshape(1, HD2)
    pt, pg, pi, cst = pl.pallas_call(
        _proj_small_body,
        out_shape=(
            jax.ShapeDtypeStruct((emb_test.shape[0], HD2), jnp.float32),
            jax.ShapeDtypeStruct((emb_tag.shape[0], HD2), jnp.float32),
            jax.ShapeDtypeStruct((8, HD2), jnp.float32),
            jax.ShapeDtypeStruct((1, 288), jnp.float32),
        ),
    )(emb_test, emb_tag, ei_pad, W_comb, row(b_comb), row(g_comb),
      row(be_comb), W_cont, row(b_cont), row(g_cont), row(be_cont))

    mesh = plsc.VectorSubcoreMesh(core_axis_name="c", subcore_axis_name="s",
                                  num_cores=_NC, num_subcores=_NS)
    sc = pl.kernel(
        _sc_body,
        out_type=jax.ShapeDtypeStruct((BL, 96), jnp.float32),
        mesh=mesh,
        compiler_params=pltpu.CompilerParams(needs_layout_passes=False,
                                             use_tc_tiling_on_sc=False),
        scratch_types=[
            pltpu.VMEM((CH,), jnp.int32),
            pltpu.VMEM((CH,), jnp.int32),
            pltpu.VMEM((CH,), jnp.int32),
            pltpu.VMEM((CH,), jnp.int32),
            pltpu.VMEM((CH,), jnp.float32),
            pltpu.VMEM((CH, HD2), jnp.float32),
            pltpu.VMEM((CH, HD2), jnp.float32),
            pltpu.VMEM((CH, HD2), jnp.float32),
            pltpu.VMEM((CH, HD2), jnp.float32),
            pltpu.VMEM((CH, 96), jnp.float32),
            pltpu.VMEM((288,), jnp.float32),
            pltpu.SemaphoreType.DMA,
        ],
    )
    x2 = sc(question.reshape(BL).astype(jnp.int32),
            test.reshape(BL).astype(jnp.int32),
            tag.reshape(BL).astype(jnp.int32),
            interaction.reshape(BL).astype(jnp.int32),
            elapsed.reshape(BL).astype(jnp.float32),
            pq, pt, pg, pi, cst.reshape(288))
    return x2.reshape(B, L, 96)
